# baseline (speedup 1.0000x reference)
import functools

import jax
import jax.numpy as jnp
from jax import lax
from jax.experimental import pallas as pl
from jax.experimental.pallas import tpu as pltpu
from jax.experimental.pallas import tpu_sc as plsc

B = 4
S = 2048
D = 1024
LANES = 16

_info = plsc.get_sparse_core_info()
NC = _info.num_cores
NS = _info.num_subcores
NW = NC * NS

S_PER_W = S // NW
CHUNK = 16
NSUB = S_PER_W // CHUNK
NCHUNK = B * NSUB
VPR = D // LANES
NBUF = 3


def _body(ids_hbm, tok_hbm, pos_hbm, out_hbm,
          idx_v, pos_v, tok_a, tok_b, tok_c,
          psem, gsem_a, gsem_b, gsem_c, ssem_a, ssem_b, ssem_c):
    c = lax.axis_index("c")
    s = lax.axis_index("s")
    wid = s * NC + c
    s0 = wid * S_PER_W

    toks = (tok_a, tok_b, tok_c)
    gsems = (gsem_a, gsem_b, gsem_c)
    ssems = (ssem_a, ssem_b, ssem_c)

    pos_cp = pltpu.async_copy(pos_hbm.at[pl.ds(s0, S_PER_W)], pos_v, psem)
    for b in range(B):
        pltpu.sync_copy(ids_hbm.at[pl.ds(b * S + s0, S_PER_W)], idx_v.at[b])

    def chunk_coords(k):
        b, sub = divmod(k, NSUB)
        return b, sub

    def issue_gather(k):
        p = k % NBUF
        b, sub = chunk_coords(k)
        return pltpu.async_copy(
            tok_hbm.at[idx_v.at[b, pl.ds(sub * CHUNK, CHUNK)]],
            toks[p], gsems[p])

    gathers = {}
    stores = {}
    gathers[0] = issue_gather(0)
    gathers[1] = issue_gather(1)

    pos_cp.wait()

    for k in range(NCHUNK):
        p = k % NBUF
        b, sub = chunk_coords(k)
        gathers[k].wait()
        if k + 2 < NCHUNK:
            if k - 1 in stores:
                stores[k - 1].wait()
            gathers[k + 2] = issue_gather(k + 2)
        tok = toks[p]
        row0 = sub * CHUNK

        def add_row(r, _):
            for j in range(VPR):
                sl = pl.ds(j * LANES, LANES)
                plsc.addupdate(tok.at[r, sl], pos_v[row0 + r, sl])
            return 0

        lax.fori_loop(0, CHUNK, add_row, 0)
        stores[k] = pltpu.async_copy(
            tok, out_hbm.at[b, pl.ds(s0 + row0, CHUNK)], ssems[p])

    stores[NCHUNK - 3].wait()
    stores[NCHUNK - 2].wait()
    stores[NCHUNK - 1].wait()


_sc_call = functools.partial(
    pl.kernel,
    out_type=jax.ShapeDtypeStruct((B, S, D), jnp.float32),
    mesh=plsc.VectorSubcoreMesh(core_axis_name="c", subcore_axis_name="s"),
    scratch_types=[
        pltpu.VMEM((B, S_PER_W), jnp.int32),
        pltpu.VMEM((S_PER_W, D), jnp.float32),
        pltpu.VMEM((CHUNK, D), jnp.float32),
        pltpu.VMEM((CHUNK, D), jnp.float32),
        pltpu.VMEM((CHUNK, D), jnp.float32),
        pltpu.SemaphoreType.DMA,
        pltpu.SemaphoreType.DMA,
        pltpu.SemaphoreType.DMA,
        pltpu.SemaphoreType.DMA,
        pltpu.SemaphoreType.DMA,
        pltpu.SemaphoreType.DMA,
        pltpu.SemaphoreType.DMA,
    ],
)(_body)


def kernel(input_ids, embed_tokens_weight, embed_positions_weight):
    ids_flat = input_ids.reshape(-1).astype(jnp.int32)
    pos_sliced = lax.slice(embed_positions_weight, (2, 0), (S + 2, D))
    return _sc_call(ids_flat, embed_tokens_weight, pos_sliced)


# speedup vs baseline: 1.1040x; 1.1040x over previous
import functools

import jax
import jax.numpy as jnp
from jax import lax
from jax.experimental import pallas as pl
from jax.experimental.pallas import tpu as pltpu
from jax.experimental.pallas import tpu_sc as plsc

B = 4
S = 2048
D = 1024
LANES = 16

_info = plsc.get_sparse_core_info()
NC = _info.num_cores
NS = _info.num_subcores
NW = NC * NS

S_PER_W = S // NW
CHUNK = 16
NSUB = S_PER_W // CHUNK
NCHUNK = B * NSUB
VPR = D // LANES
NBUF = 3


def _body(ids_hbm, tok_hbm, pos_hbm, out_hbm,
          idx_v, pos_v, tok_a, tok_b, tok_c,
          psem, gsem_a, gsem_b, gsem_c, ssem_a, ssem_b, ssem_c):
    c = lax.axis_index("c")
    s = lax.axis_index("s")
    wid = s * NC + c
    s0 = wid * S_PER_W

    toks = (tok_a, tok_b, tok_c)
    gsems = (gsem_a, gsem_b, gsem_c)
    ssems = (ssem_a, ssem_b, ssem_c)

    pos_cp = pltpu.async_copy(pos_hbm.at[pl.ds(s0, S_PER_W)], pos_v, psem)
    for b in range(B):
        pltpu.sync_copy(ids_hbm.at[pl.ds(b * S + s0, S_PER_W)], idx_v.at[b])

    def chunk_coords(k):
        b, sub = divmod(k, NSUB)
        return b, sub

    def issue_gather(k):
        p = k % NBUF
        b, sub = chunk_coords(k)
        return pltpu.async_copy(
            tok_hbm.at[idx_v.at[b, pl.ds(sub * CHUNK, CHUNK)]],
            toks[p], gsems[p])

    gathers = {}
    stores = {}
    gathers[0] = issue_gather(0)
    gathers[1] = issue_gather(1)

    pos_cp.wait()

    for k in range(NCHUNK):
        p = k % NBUF
        b, sub = chunk_coords(k)
        gathers[k].wait()
        tok = toks[p]
        row0 = sub * CHUNK

        def add_row(r, _):
            for j in range(VPR):
                sl = pl.ds(j * LANES, LANES)
                plsc.addupdate(tok.at[r, sl], pos_v[row0 + r, sl])
            return 0

        lax.fori_loop(0, CHUNK, add_row, 0)
        stores[k] = pltpu.async_copy(
            tok, out_hbm.at[b, pl.ds(s0 + row0, CHUNK)], ssems[p])
        if k + 2 < NCHUNK:
            if k - 1 in stores:
                stores[k - 1].wait()
            gathers[k + 2] = issue_gather(k + 2)

    stores[NCHUNK - 3].wait()
    stores[NCHUNK - 2].wait()
    stores[NCHUNK - 1].wait()


_sc_call = functools.partial(
    pl.kernel,
    out_type=jax.ShapeDtypeStruct((B, S, D), jnp.float32),
    mesh=plsc.VectorSubcoreMesh(core_axis_name="c", subcore_axis_name="s"),
    scratch_types=[
        pltpu.VMEM((B, S_PER_W), jnp.int32),
        pltpu.VMEM((S_PER_W, D), jnp.float32),
        pltpu.VMEM((CHUNK, D), jnp.float32),
        pltpu.VMEM((CHUNK, D), jnp.float32),
        pltpu.VMEM((CHUNK, D), jnp.float32),
        pltpu.SemaphoreType.DMA,
        pltpu.SemaphoreType.DMA,
        pltpu.SemaphoreType.DMA,
        pltpu.SemaphoreType.DMA,
        pltpu.SemaphoreType.DMA,
        pltpu.SemaphoreType.DMA,
        pltpu.SemaphoreType.DMA,
    ],
)(_body)


def kernel(input_ids, embed_tokens_weight, embed_positions_weight):
    ids_flat = input_ids.reshape(-1).astype(jnp.int32)
    pos_sliced = lax.slice(embed_positions_weight, (2, 0), (S + 2, D))
    return _sc_call(ids_flat, embed_tokens_weight, pos_sliced)


# speedup vs baseline: 1.1494x; 1.0411x over previous
import functools

import jax
import jax.numpy as jnp
from jax import lax
from jax.experimental import pallas as pl
from jax.experimental.pallas import tpu as pltpu
from jax.experimental.pallas import tpu_sc as plsc

B = 4
S = 2048
D = 1024
OFFSET = 2
LANES = 16

_info = plsc.get_sparse_core_info()
NC = _info.num_cores
NS = _info.num_subcores
NW = NC * NS

S_PER_W = S // NW
CHUNK = 16
NSUB = S_PER_W // CHUNK
NCHUNK = B * NSUB
VPR = D // LANES
NBUF = 3


def _body(ids_hbm, tok_hbm, pos_hbm, out_hbm,
          idx_v, pidx_v, pos_v, tok_a, tok_b, tok_c,
          psem, gsem_a, gsem_b, gsem_c, ssem_a, ssem_b, ssem_c):
    c = lax.axis_index("c")
    s = lax.axis_index("s")
    wid = s * NC + c
    s0 = wid * S_PER_W

    toks = (tok_a, tok_b, tok_c)
    gsems = (gsem_a, gsem_b, gsem_c)
    ssems = (ssem_a, ssem_b, ssem_c)

    for t in range(S_PER_W // LANES):
        pidx_v[pl.ds(t * LANES, LANES)] = (
            lax.iota(jnp.int32, LANES) + (s0 + OFFSET + t * LANES))
    pos_cp = pltpu.async_copy(pos_hbm.at[pidx_v], pos_v, psem)
    for b in range(B):
        pltpu.sync_copy(ids_hbm.at[pl.ds(b * S + s0, S_PER_W)],
                        idx_v.at[pl.ds(b * S_PER_W, S_PER_W)])

    def chunk_coords(k):
        b, sub = divmod(k, NSUB)
        return b, sub

    def issue_gather(k):
        p = k % NBUF
        b, sub = chunk_coords(k)
        return pltpu.async_copy(
            tok_hbm.at[idx_v.at[pl.ds(b * S_PER_W + sub * CHUNK, CHUNK)]],
            toks[p], gsems[p])

    gathers = {}
    stores = {}
    gathers[0] = issue_gather(0)
    gathers[1] = issue_gather(1)

    pos_cp.wait()

    for k in range(NCHUNK):
        p = k % NBUF
        b, sub = chunk_coords(k)
        gathers[k].wait()
        tok = toks[p]
        row0 = sub * CHUNK

        def add_row(r, _):
            for j in range(VPR):
                sl = pl.ds(j * LANES, LANES)
                plsc.addupdate(tok.at[r, sl], pos_v[row0 + r, sl])
            return 0

        lax.fori_loop(0, CHUNK, add_row, 0)
        stores[k] = pltpu.async_copy(
            tok, out_hbm.at[b, pl.ds(s0 + row0, CHUNK)], ssems[p])
        if k + 2 < NCHUNK:
            if k - 1 in stores:
                stores[k - 1].wait()
            gathers[k + 2] = issue_gather(k + 2)

    stores[NCHUNK - 3].wait()
    stores[NCHUNK - 2].wait()
    stores[NCHUNK - 1].wait()


_sc_call = functools.partial(
    pl.kernel,
    out_type=jax.ShapeDtypeStruct((B, S, D), jnp.float32),
    mesh=plsc.VectorSubcoreMesh(core_axis_name="c", subcore_axis_name="s"),
    scratch_types=[
        pltpu.VMEM((B * S_PER_W,), jnp.int32),
        pltpu.VMEM((S_PER_W,), jnp.int32),
        pltpu.VMEM((S_PER_W, D), jnp.float32),
        pltpu.VMEM((CHUNK, D), jnp.float32),
        pltpu.VMEM((CHUNK, D), jnp.float32),
        pltpu.VMEM((CHUNK, D), jnp.float32),
        pltpu.SemaphoreType.DMA,
        pltpu.SemaphoreType.DMA,
        pltpu.SemaphoreType.DMA,
        pltpu.SemaphoreType.DMA,
        pltpu.SemaphoreType.DMA,
        pltpu.SemaphoreType.DMA,
        pltpu.SemaphoreType.DMA,
    ],
)(_body)


def kernel(input_ids, embed_tokens_weight, embed_positions_weight):
    ids_flat = input_ids.reshape(-1)
    return _sc_call(ids_flat, embed_tokens_weight, embed_positions_weight)


# speedup vs baseline: 1.3547x; 1.1786x over previous
import functools

import jax
import jax.numpy as jnp
from jax import lax
from jax.experimental import pallas as pl
from jax.experimental.pallas import tpu as pltpu
from jax.experimental.pallas import tpu_sc as plsc

B = 4
S = 2048
D = 1024
OFFSET = 2
LANES = 16

_info = plsc.get_sparse_core_info()
NC = _info.num_cores
NS = _info.num_subcores
NW = NC * NS

S_PER_W = S // NW
CHUNK = 16
NSUB = S_PER_W // CHUNK
NCHUNK = B * NSUB
VPR = D // LANES
NBUF = 3


def _body(ids_hbm, tok_hbm, pos_hbm, out_hbm,
          idx_v, pidx_v, pos_v, tok_a, tok_b, tok_c,
          psem, gsem_a, gsem_b, gsem_c, ssem_a, ssem_b, ssem_c):
    c = lax.axis_index("c")
    s = lax.axis_index("s")
    wid = s * NC + c
    s0 = wid * S_PER_W

    toks = (tok_a, tok_b, tok_c)
    gsems = (gsem_a, gsem_b, gsem_c)
    ssems = (ssem_a, ssem_b, ssem_c)

    for t in range(S_PER_W // LANES):
        pidx_v[pl.ds(t * LANES, LANES)] = (
            lax.iota(jnp.int32, LANES) + (s0 + OFFSET + t * LANES))
    pos_cp = pltpu.async_copy(pos_hbm.at[pidx_v], pos_v, psem)
    for b in range(B):
        pltpu.sync_copy(ids_hbm.at[pl.ds(b * S + s0, S_PER_W)],
                        idx_v.at[pl.ds(b * S_PER_W, S_PER_W)])

    def chunk_coords(k):
        b, sub = divmod(k, NSUB)
        return b, sub

    def issue_gather(k):
        p = k % NBUF
        b, sub = chunk_coords(k)
        return pltpu.async_copy(
            tok_hbm.at[idx_v.at[pl.ds(b * S_PER_W + sub * CHUNK, CHUNK)]],
            toks[p], gsems[p])

    gathers = {}
    stores = {}
    gathers[0] = issue_gather(0)
    gathers[1] = issue_gather(1)

    pos_cp.wait()

    for k in range(NCHUNK):
        p = k % NBUF
        b, sub = chunk_coords(k)
        gathers[k].wait()
        tok = toks[p]
        row0 = sub * CHUNK

        def add_row(r, _):
            for g in range(VPR // 8):
                sls = [pl.ds((g * 8 + j) * LANES, LANES) for j in range(8)]
                vals = [pos_v[row0 + r, sl] for sl in sls]
                for sl, v in zip(sls, vals):
                    plsc.addupdate(tok.at[r, sl], v)
            return 0

        lax.fori_loop(0, CHUNK, add_row, 0)
        stores[k] = pltpu.async_copy(
            tok, out_hbm.at[b, pl.ds(s0 + row0, CHUNK)], ssems[p])
        if k + 2 < NCHUNK:
            if k - 1 in stores:
                stores[k - 1].wait()
            gathers[k + 2] = issue_gather(k + 2)

    stores[NCHUNK - 3].wait()
    stores[NCHUNK - 2].wait()
    stores[NCHUNK - 1].wait()


_sc_call = functools.partial(
    pl.kernel,
    out_type=jax.ShapeDtypeStruct((B, S, D), jnp.float32),
    mesh=plsc.VectorSubcoreMesh(core_axis_name="c", subcore_axis_name="s"),
    scratch_types=[
        pltpu.VMEM((B * S_PER_W,), jnp.int32),
        pltpu.VMEM((S_PER_W,), jnp.int32),
        pltpu.VMEM((S_PER_W, D), jnp.float32),
        pltpu.VMEM((CHUNK, D), jnp.float32),
        pltpu.VMEM((CHUNK, D), jnp.float32),
        pltpu.VMEM((CHUNK, D), jnp.float32),
        pltpu.SemaphoreType.DMA,
        pltpu.SemaphoreType.DMA,
        pltpu.SemaphoreType.DMA,
        pltpu.SemaphoreType.DMA,
        pltpu.SemaphoreType.DMA,
        pltpu.SemaphoreType.DMA,
        pltpu.SemaphoreType.DMA,
    ],
)(_body)


def kernel(input_ids, embed_tokens_weight, embed_positions_weight):
    ids_flat = input_ids.reshape(-1)
    return _sc_call(ids_flat, embed_tokens_weight, embed_positions_weight)


# speedup vs baseline: 1.4685x; 1.0840x over previous
import functools

import jax
import jax.numpy as jnp
from jax import lax
from jax.experimental import pallas as pl
from jax.experimental.pallas import tpu as pltpu
from jax.experimental.pallas import tpu_sc as plsc

B = 4
S = 2048
D = 1024
OFFSET = 2
LANES = 16

_info = plsc.get_sparse_core_info()
NC = _info.num_cores
NS = _info.num_subcores
NW = NC * NS

S_PER_W = S // NW
CHUNK = 16
NSUB = S_PER_W // CHUNK
NCHUNK = B * NSUB
VPR = D // LANES
NBUF = 3


def _body(ids_hbm, tok_hbm, pos_hbm, out_hbm,
          idx_v, pidx_v, pos_v, tok_a, tok_b, tok_c,
          psem, isem, gsem_a, gsem_b, gsem_c, ssem_a, ssem_b, ssem_c):
    c = lax.axis_index("c")
    s = lax.axis_index("s")
    wid = s * NC + c
    s0 = wid * S_PER_W

    toks = (tok_a, tok_b, tok_c)
    gsems = (gsem_a, gsem_b, gsem_c)
    ssems = (ssem_a, ssem_b, ssem_c)

    for t in range(S_PER_W // LANES):
        pidx_v[pl.ds(t * LANES, LANES)] = (
            lax.iota(jnp.int32, LANES) + (s0 + OFFSET + t * LANES))
    pos_cp = pltpu.async_copy(pos_hbm.at[pidx_v], pos_v, psem)
    idx_cps = [
        pltpu.async_copy(ids_hbm.at[pl.ds(b * S + s0, S_PER_W)],
                         idx_v.at[pl.ds(b * S_PER_W, S_PER_W)], isem)
        for b in range(B)
    ]
    for cp in idx_cps:
        cp.wait()

    def issue_gather(k, p):
        pltpu.async_copy(tok_hbm.at[idx_v.at[pl.ds(k * CHUNK, CHUNK)]],
                         toks[p], gsems[p])

    issue_gather(0, 0)
    issue_gather(1, 1)
    pos_cp.wait()

    def wait_gather(p):
        pltpu.make_async_copy(tok_hbm.at[idx_v.at[pl.ds(0, CHUNK)]],
                              toks[p], gsems[p]).wait()

    def wait_store(p):
        pltpu.make_async_copy(toks[p], out_hbm.at[0, pl.ds(0, CHUNK)],
                              ssems[p]).wait()

    def run_group(g, _):
        for p in range(NBUF):
            k = g * NBUF + p

            @pl.when(k < NCHUNK)
            def _():
                wait_gather(p)
                tok = toks[p]
                row0 = (k % NSUB) * CHUNK

                def add_row(r, __):
                    for gg in range(VPR // 8):
                        sls = [pl.ds((gg * 8 + j) * LANES, LANES)
                               for j in range(8)]
                        vals = [pos_v[row0 + r, sl] for sl in sls]
                        for sl, v in zip(sls, vals):
                            plsc.addupdate(tok.at[r, sl], v)
                    return 0

                lax.fori_loop(0, CHUNK, add_row, 0)
                pltpu.async_copy(
                    tok, out_hbm.at[k // NSUB, pl.ds(s0 + row0, CHUNK)],
                    ssems[p])

            @pl.when(jnp.logical_and(k >= 1, k + 2 < NCHUNK))
            def _():
                wait_store((p + 2) % NBUF)

            @pl.when(k + 2 < NCHUNK)
            def _():
                issue_gather(k + 2, (p + 2) % NBUF)
        return 0

    lax.fori_loop(0, (NCHUNK + NBUF - 1) // NBUF, run_group, 0)

    wait_store((NCHUNK - 3) % NBUF)
    wait_store((NCHUNK - 2) % NBUF)
    wait_store((NCHUNK - 1) % NBUF)


_sc_call = functools.partial(
    pl.kernel,
    out_type=jax.ShapeDtypeStruct((B, S, D), jnp.float32),
    mesh=plsc.VectorSubcoreMesh(core_axis_name="c", subcore_axis_name="s"),
    scratch_types=[
        pltpu.VMEM((B * S_PER_W,), jnp.int32),
        pltpu.VMEM((S_PER_W,), jnp.int32),
        pltpu.VMEM((S_PER_W, D), jnp.float32),
        pltpu.VMEM((CHUNK, D), jnp.float32),
        pltpu.VMEM((CHUNK, D), jnp.float32),
        pltpu.VMEM((CHUNK, D), jnp.float32),
        pltpu.SemaphoreType.DMA,
        pltpu.SemaphoreType.DMA,
        pltpu.SemaphoreType.DMA,
        pltpu.SemaphoreType.DMA,
        pltpu.SemaphoreType.DMA,
        pltpu.SemaphoreType.DMA,
        pltpu.SemaphoreType.DMA,
        pltpu.SemaphoreType.DMA,
    ],
)(_body)


def kernel(input_ids, embed_tokens_weight, embed_positions_weight):
    ids_flat = input_ids.reshape(-1)
    return _sc_call(ids_flat, embed_tokens_weight, embed_positions_weight)


# speedup vs baseline: 1.4776x; 1.0062x over previous
import functools

import jax
import jax.numpy as jnp
from jax import lax
from jax.experimental import pallas as pl
from jax.experimental.pallas import tpu as pltpu
from jax.experimental.pallas import tpu_sc as plsc

B = 4
S = 2048
D = 1024
OFFSET = 2
LANES = 16

_info = plsc.get_sparse_core_info()
NC = _info.num_cores
NS = _info.num_subcores
NW = NC * NS

S_PER_W = S // NW
CHUNK = 16
NSUB = S_PER_W // CHUNK
NCHUNK = B * NSUB
VPR = D // LANES
NBUF = 3


def _body(ids_hbm, tok_hbm, pos_hbm, out_hbm,
          idx_v, pidx_v, pos_v, tok_a, tok_b, tok_c,
          psem, gsem_a, gsem_b, gsem_c, ssem_a, ssem_b, ssem_c):
    c = lax.axis_index("c")
    s = lax.axis_index("s")
    wid = s * NC + c
    s0 = wid * S_PER_W

    toks = (tok_a, tok_b, tok_c)
    gsems = (gsem_a, gsem_b, gsem_c)
    ssems = (ssem_a, ssem_b, ssem_c)

    for t in range(S_PER_W // LANES):
        pidx_v[pl.ds(t * LANES, LANES)] = (
            lax.iota(jnp.int32, LANES) + (s0 + OFFSET + t * LANES))
    pos_cp = pltpu.async_copy(pos_hbm.at[pidx_v], pos_v, psem)
    col0 = (wid // 2) * (2 * S_PER_W)
    coff = (wid % 2) * S_PER_W
    pltpu.sync_copy(ids_hbm.at[:, pl.ds(col0, 2 * S_PER_W)], idx_v)

    def issue_gather(k, p):
        b = k // NSUB
        sub = k % NSUB
        pltpu.async_copy(
            tok_hbm.at[idx_v.at[b, pl.ds(coff + sub * CHUNK, CHUNK)]],
            toks[p], gsems[p])

    issue_gather(0, 0)
    issue_gather(1, 1)
    pos_cp.wait()

    def wait_gather(p):
        pltpu.make_async_copy(tok_hbm.at[idx_v.at[0, pl.ds(0, CHUNK)]],
                              toks[p], gsems[p]).wait()

    def wait_store(p):
        pltpu.make_async_copy(toks[p], out_hbm.at[0, pl.ds(0, CHUNK)],
                              ssems[p]).wait()

    def run_group(g, _):
        for p in range(NBUF):
            k = g * NBUF + p

            @pl.when(k < NCHUNK)
            def _():
                wait_gather(p)
                tok = toks[p]
                row0 = (k % NSUB) * CHUNK

                def add_row(r, __):
                    for gg in range(VPR // 8):
                        sls = [pl.ds((gg * 8 + j) * LANES, LANES)
                               for j in range(8)]
                        vals = [pos_v[row0 + r, sl] for sl in sls]
                        for sl, v in zip(sls, vals):
                            plsc.addupdate(tok.at[r, sl], v)
                    return 0

                lax.fori_loop(0, CHUNK, add_row, 0)
                pltpu.async_copy(
                    tok, out_hbm.at[k // NSUB, pl.ds(s0 + row0, CHUNK)],
                    ssems[p])

            @pl.when(jnp.logical_and(k >= 1, k + 2 < NCHUNK))
            def _():
                wait_store((p + 2) % NBUF)

            @pl.when(k + 2 < NCHUNK)
            def _():
                issue_gather(k + 2, (p + 2) % NBUF)
        return 0

    lax.fori_loop(0, (NCHUNK + NBUF - 1) // NBUF, run_group, 0)

    wait_store((NCHUNK - 3) % NBUF)
    wait_store((NCHUNK - 2) % NBUF)
    wait_store((NCHUNK - 1) % NBUF)


_sc_call = functools.partial(
    pl.kernel,
    out_type=jax.ShapeDtypeStruct((B, S, D), jnp.float32),
    mesh=plsc.VectorSubcoreMesh(core_axis_name="c", subcore_axis_name="s"),
    scratch_types=[
        pltpu.VMEM((B, 2 * S_PER_W), jnp.int32),
        pltpu.VMEM((S_PER_W,), jnp.int32),
        pltpu.VMEM((S_PER_W, D), jnp.float32),
        pltpu.VMEM((CHUNK, D), jnp.float32),
        pltpu.VMEM((CHUNK, D), jnp.float32),
        pltpu.VMEM((CHUNK, D), jnp.float32),
        pltpu.SemaphoreType.DMA,
        pltpu.SemaphoreType.DMA,
        pltpu.SemaphoreType.DMA,
        pltpu.SemaphoreType.DMA,
        pltpu.SemaphoreType.DMA,
        pltpu.SemaphoreType.DMA,
        pltpu.SemaphoreType.DMA,
    ],
)(_body)


def kernel(input_ids, embed_tokens_weight, embed_positions_weight):
    return _sc_call(input_ids, embed_tokens_weight, embed_positions_weight)
